# Initial kernel scaffold; baseline (speedup 1.0000x reference)
import jax
import jax.numpy as jnp
from jax.experimental import pallas as pl


def kernel(x, edge_index, W_in, b_in, Wg1, bg1, g1, be1, Wg2, bg2, g2, be2, W_out, b_out):
    raise NotImplementedError("write your pallas kernel here")


# speedup vs baseline: 1.6472x; 1.6472x over previous
import jax
import jax.numpy as jnp
from jax import lax
from jax.experimental import pallas as pl
from jax.experimental.pallas import tpu as pltpu
from jax.experimental.pallas import tpu_sc as plsc

N = 10000
NP = 10240
HN = NP // 2
D = 128
E = 320000
NC = 2
NS = 16
NW = NC * NS
CG = 8
CW = 16
CHK = 128
JI = 16
NCH2 = 80
EPH = NCH2 * JI * CHK
EP = EPH * 2
EPW = EP // NW
AW = HN * CW
R = 512
_grid = (NP // R,)

_mesh = plsc.VectorSubcoreMesh(
    core_axis_name="c", subcore_axis_name="s", num_cores=NC, num_subcores=NS)


def _deg_body(dst_hbm, dg_hbm, idx_d, acc):
    c = lax.axis_index("c")
    s = lax.axis_index("s")
    w = s * NC + c
    def z(i, _):
        acc[pl.ds(i * 16, 16)] = jnp.zeros((16,), jnp.float32)
        return 0
    lax.fori_loop(0, (NP + 16) // 16, z, 0)
    pltpu.sync_copy(dst_hbm.at[w], idx_d)
    io = lax.iota(jnp.int32, 16)
    one0 = jnp.where(io == 0, 1.0, 0.0).astype(jnp.float32)
    def body(i, _):
        d16 = idx_d[i // 8, pl.ds((i % 8) * 16, 16)]
        for e in range(16):
            d = d16[e]
            acc[pl.ds(d, 16)] = acc[pl.ds(d, 16)] + one0
        return 0
    lax.fori_loop(0, EPW // 16, body, 0)
    pltpu.sync_copy(acc.at[pl.ds(0, NP)], dg_hbm.at[w])


_deg_call = pl.kernel(
    _deg_body,
    out_type=jax.ShapeDtypeStruct((NW, NP), jnp.float32),
    mesh=_mesh,
    scratch_types=[
        pltpu.VMEM((EPW // CHK, CHK), jnp.int32),
        pltpu.VMEM((NP + 16,), jnp.float32),
    ],
)


def _msg_body(src_hbm, dst_hbm, g_hbm, p_hbm, idx_s, idx_d, rows, acc, sem):
    c = lax.axis_index("c")
    s = lax.axis_index("s")
    w = s * NC + c
    cg = lax.rem(w, CG)
    nh = lax.rem(w // CG, 2)
    eh = w // (CG * 2)
    lo = nh * HN
    def z(i, _):
        acc[pl.ds(i * 16, 16)] = jnp.zeros((16,), jnp.float32)
        return 0
    lax.fori_loop(0, (AW + 32) // 16, z, 0)
    col = cg * CW
    def outer(jo, _):
        pltpu.sync_copy(src_hbm.at[eh, jo], idx_s)
        pltpu.sync_copy(dst_hbm.at[eh, jo], idx_d)
        def inner(ji, _):
            pltpu.async_copy(g_hbm.at[idx_s.at[ji]], rows, sem).wait()
            for q in range(CHK // 16):
                d16 = idx_d[ji, pl.ds(q * 16, 16)]
                for e in range(16):
                    d = d16[e]
                    dloc = d - lo
                    ok = (dloc >= 0) & (dloc < HN)
                    off = jnp.where(ok, dloc * CW, AW)
                    v = rows[q * 16 + e, pl.ds(col, CW)]
                    acc[pl.ds(off, 16)] = acc[pl.ds(off, 16)] + v
            return 0
        lax.fori_loop(0, JI, inner, 0)
        return 0
    lax.fori_loop(0, NCH2, outer, 0)
    for k in range(CG):
        pltpu.sync_copy(
            acc.at[pl.ds(k * (AW // CG), AW // CG)],
            p_hbm.at[eh, cg, pl.ds(nh * AW + k * (AW // CG), AW // CG)])


_msg_call = pl.kernel(
    _msg_body,
    out_type=jax.ShapeDtypeStruct((2, CG, NP * CW), jnp.float32),
    mesh=_mesh,
    scratch_types=[
        pltpu.VMEM((JI, CHK), jnp.int32),
        pltpu.VMEM((JI, CHK), jnp.int32),
        pltpu.VMEM((CHK, D), jnp.float32),
        pltpu.VMEM((AW + 32,), jnp.float32),
        pltpu.SemaphoreType.DMA,
    ],
)


def _dinv_of(dg_blk):
    return lax.rsqrt(jnp.sum(dg_blk, axis=0).reshape(-1, 1) + 1.0)


def _psum_of(p0_blk, p1_blk):
    q = p0_blk + p1_blk
    return jnp.concatenate([q[k] for k in range(CG)], axis=-1)


def _pro_body(x_r, win_r, bin_r, wg_r, dg_r, h_o, g_o):
    hv = jnp.dot(x_r[:], win_r[:], preferred_element_type=jnp.float32) + bin_r[:]
    dinv = _dinv_of(dg_r[:])
    h_o[:] = hv
    g_o[:] = jnp.dot(hv, wg_r[:], preferred_element_type=jnp.float32) * dinv


def _post_core(psum, g, dinv, gam, bet, bconv, hprev):
    sacc = (psum + g) * dinv + bconv
    mu = jnp.mean(sacc, axis=1, keepdims=True)
    var = jnp.mean((sacc - mu) ** 2, axis=1, keepdims=True)
    ln = (sacc - mu) * lax.rsqrt(var + 1e-5) * gam + bet
    return jnp.maximum(ln, 0.0) + hprev


def _mid_body(p0_r, p1_r, g_r, dg_r, h_r, gam_r, bet_r, bc_r, w_r, h_o, g_o):
    dinv = _dinv_of(dg_r[:])
    hn = _post_core(_psum_of(p0_r[:], p1_r[:]), g_r[:], dinv, gam_r[:],
                    bet_r[:], bc_r[:], h_r[:])
    h_o[:] = hn
    g_o[:] = jnp.dot(hn, w_r[:], preferred_element_type=jnp.float32) * dinv


def _fin_body(p0_r, p1_r, g_r, dg_r, h_r, gam_r, bet_r, bc_r, w_r, bw_r,
              out_o):
    dinv = _dinv_of(dg_r[:])
    hn = _post_core(_psum_of(p0_r[:], p1_r[:]), g_r[:], dinv, gam_r[:],
                    bet_r[:], bc_r[:], h_r[:])
    out_o[:] = jnp.dot(hn, w_r[:], preferred_element_type=jnp.float32) + bw_r[:]


_bs_row = pl.BlockSpec((R, D), lambda i: (i, 0))
_bs_deg = pl.BlockSpec((NW, R), lambda i: (0, i))
_bs_p = pl.BlockSpec((CG, R, CW), lambda i: (0, i, 0))
_bs_mat = pl.BlockSpec((D, D), lambda i: (0, 0))
_bs_vec = pl.BlockSpec((1, D), lambda i: (0, 0))
_row_out = jax.ShapeDtypeStruct((NP, D), jnp.float32)

_pro_call = pl.pallas_call(
    _pro_body, grid=_grid,
    in_specs=[_bs_row, _bs_mat, _bs_vec, _bs_mat, _bs_deg],
    out_specs=[_bs_row, _bs_row],
    out_shape=[_row_out, _row_out],
)

_mid_call = pl.pallas_call(
    _mid_body, grid=_grid,
    in_specs=[_bs_p, _bs_p, _bs_row, _bs_deg, _bs_row,
              _bs_vec, _bs_vec, _bs_vec, _bs_mat],
    out_specs=[_bs_row, _bs_row],
    out_shape=[_row_out, _row_out],
)

_fin_call = pl.pallas_call(
    _fin_body, grid=_grid,
    in_specs=[_bs_p, _bs_p, _bs_row, _bs_deg, _bs_row,
              _bs_vec, _bs_vec, _bs_vec, _bs_mat, _bs_vec],
    out_specs=_bs_row,
    out_shape=_row_out,
)


def kernel(x, edge_index, W_in, b_in, Wg1, bg1, g1, be1, Wg2, bg2, g2, be2,
           W_out, b_out):
    ei = edge_index.astype(jnp.int32)
    npad = EP - E
    pad_src = (jnp.arange(npad, dtype=jnp.int32) * 61) % N
    pad_dst = N + jnp.arange(npad, dtype=jnp.int32) % (NP - N)
    srcp = jnp.concatenate([ei[0], pad_src])
    dstp = jnp.concatenate([ei[1], pad_dst])
    src_m = srcp.reshape(2, NCH2, JI, CHK)
    dst_m = dstp.reshape(2, NCH2, JI, CHK)
    dst_d = dstp.reshape(NW, EPW // CHK, CHK)
    xp = jnp.zeros((NP, D), jnp.float32).at[:N].set(x)

    dgp = _deg_call(dst_d)
    hh, gm1 = _pro_call(xp, W_in, b_in.reshape(1, D), Wg1, dgp)
    p = _msg_call(src_m, dst_m, gm1).reshape(2, CG, NP, CW)
    h2, gm2 = _mid_call(p[0], p[1], gm1, dgp, hh, g1.reshape(1, D),
                        be1.reshape(1, D), bg1.reshape(1, D), Wg2)
    q = _msg_call(src_m, dst_m, gm2).reshape(2, CG, NP, CW)
    out = _fin_call(q[0], q[1], gm2, dgp, h2, g2.reshape(1, D),
                    be2.reshape(1, D), bg2.reshape(1, D), W_out,
                    b_out.reshape(1, D))
    return out[:N]
